# Initial kernel scaffold; baseline (speedup 1.0000x reference)
import functools

import jax
import jax.numpy as jnp
from jax import lax
from jax.experimental import pallas as pl
from jax.experimental.pallas import tpu as pltpu
from jax.experimental.pallas import tpu_sc as plsc

N = 10000
E = 320000
D = 128
DEGW = 16

NSC = 2
NSUB = 16
NW = NSC * NSUB

CH = 128
PER_W = 10240
CHUNKS = PER_W // CH
E_PAD = PER_W * NW
PAD_E = E_PAD - E

N_PAD = 10240
STRIPE = N_PAD // NSUB

R = 1000


def _make_sc_agg(with_deg: bool):
    mesh = plsc.VectorSubcoreMesh(core_axis_name="c", subcore_axis_name="s")

    out_type = [jax.ShapeDtypeStruct((NSC, N_PAD, D), jnp.float32)]
    scratch = [
        pltpu.VMEM((CH,), jnp.int32),
        pltpu.VMEM((CH,), jnp.int32),
        pltpu.VMEM((CH, D), jnp.float32),
        pltpu.VMEM_SHARED((N_PAD, D), jnp.float32),
        pltpu.SemaphoreType.DMA,
    ]
    if with_deg:
        out_type.append(jax.ShapeDtypeStruct((NSC, N_PAD, DEGW), jnp.float32))
        scratch += [
            pltpu.VMEM((CH, DEGW), jnp.float32),
            pltpu.VMEM_SHARED((N_PAD, DEGW), jnp.float32)
        ]

    @functools.partial(pl.kernel, out_type=tuple(out_type), mesh=mesh,
                       scratch_types=scratch)
    def sc_agg(*refs):
        if with_deg:
            (table_hbm, src_hbm, dst_hbm, zrows_hbm, zdeg_hbm, ones_hbm,
             out_hbm, dout_hbm,
             src_v, dst_v, rows_v, acc, sem, ones_v, dacc) = refs
        else:
            (table_hbm, src_hbm, dst_hbm, zrows_hbm,
             out_hbm,
             src_v, dst_v, rows_v, acc, sem) = refs

        c = lax.axis_index("c")
        s = lax.axis_index("s")

        pltpu.sync_copy(zrows_hbm, acc.at[pl.ds(s * STRIPE, STRIPE)])
        if with_deg:
            pltpu.sync_copy(zdeg_hbm, dacc.at[pl.ds(s * STRIPE, STRIPE)])
            pltpu.sync_copy(ones_hbm, ones_v)
        plsc.subcore_barrier()

        ebase = (c * NSUB + s) * PER_W

        @pl.loop(0, CHUNKS)
        def _(k):
            base = ebase + k * CH
            pltpu.sync_copy(src_hbm.at[pl.ds(base, CH)], src_v)
            pltpu.sync_copy(dst_hbm.at[pl.ds(base, CH)], dst_v)
            pltpu.async_copy(table_hbm.at[src_v], rows_v, sem).wait()
            pltpu.sync_copy(rows_v, acc.at[dst_v], add=True)
            if with_deg:
                pltpu.sync_copy(ones_v, dacc.at[dst_v], add=True)

        plsc.subcore_barrier()

        pltpu.sync_copy(acc.at[pl.ds(s * STRIPE, STRIPE)],
                        out_hbm.at[c, pl.ds(s * STRIPE, STRIPE)])
        if with_deg:
            pltpu.sync_copy(dacc.at[pl.ds(s * STRIPE, STRIPE)],
                            dout_hbm.at[c, pl.ds(s * STRIPE, STRIPE)])

    return sc_agg


_sc_agg_deg = _make_sc_agg(with_deg=True)
_sc_agg = _make_sc_agg(with_deg=False)


def _tc_layer(psum, dsum, x, w_l, w_r, b, relu: bool):
    def body(p0, p1, d0, d1, xr, wl, wr, br, o):
        agg = p0[0] + p1[0]
        deg = d0[0, :, 0:1] + d1[0, :, 0:1]
        rd = 1.0 / jnp.maximum(deg, 1.0)
        h = jnp.dot(agg * rd, wl[...], preferred_element_type=jnp.float32)
        h = h + jnp.dot(xr[...], wr[...], preferred_element_type=jnp.float32)
        h = h + br[...]
        o[...] = jnp.maximum(h, 0.0) if relu else h

    return pl.pallas_call(
        body,
        grid=(N // R,),
        in_specs=[
            pl.BlockSpec((1, R, D), lambda i: (0, i, 0)),
            pl.BlockSpec((1, R, D), lambda i: (1, i, 0)),
            pl.BlockSpec((1, R, DEGW), lambda i: (0, i, 0)),
            pl.BlockSpec((1, R, DEGW), lambda i: (1, i, 0)),
            pl.BlockSpec((R, D), lambda i: (i, 0)),
            pl.BlockSpec((D, D), lambda i: (0, 0)),
            pl.BlockSpec((D, D), lambda i: (0, 0)),
            pl.BlockSpec((1, D), lambda i: (0, 0)),
        ],
        out_specs=pl.BlockSpec((R, D), lambda i: (i, 0)),
        out_shape=jax.ShapeDtypeStruct((N, D), jnp.float32),
    )(psum, psum, dsum, dsum, x, w_l, w_r, b)


def kernel(x, edge_index, W1_l, W1_r, b1, W2_l, W2_r, b2):
    ei = edge_index.astype(jnp.int32)
    pad_i = jnp.arange(PAD_E, dtype=jnp.int32)
    src = jnp.concatenate([ei[0], (pad_i * 97) % N])
    dst = jnp.concatenate([ei[1], N + pad_i % (N_PAD - N)])

    zrows = jnp.zeros((STRIPE, D), jnp.float32)
    zdeg = jnp.zeros((STRIPE, DEGW), jnp.float32)
    ones = jnp.ones((CH, DEGW), jnp.float32)

    psum1, dsum = _sc_agg_deg(x, src, dst, zrows, zdeg, ones)
    h = _tc_layer(psum1, dsum, x, W1_l, W1_r, b1.reshape(1, D), relu=True)
    psum2 = _sc_agg(h, src, dst, zrows)
    out = _tc_layer(psum2, dsum, h, W2_l, W2_r, b2.reshape(1, D), relu=False)
    return out


# speedup vs baseline: 6.7691x; 6.7691x over previous
import dataclasses
import functools

import jax
import jax.numpy as jnp
from jax import lax
from jax.experimental import pallas as pl
from jax.experimental.pallas import tpu as pltpu
from jax.experimental.pallas import tpu_sc as plsc

N = 10000
E = 320000
D = 128
VL = 16

NSC = 2
NSUB = 16
NW = NSC * NSUB

CH = 128
PER_W = 10240
CHUNKS = PER_W // CH
E_PAD = PER_W * NW
PAD_E = E_PAD - E

N_PAD = 10240
STRIPE = N_PAD // NSUB
ZITER = STRIPE // CH

R = 1000


def _make_sc_agg(with_deg: bool):
    mesh = plsc.VectorSubcoreMesh(core_axis_name="c", subcore_axis_name="s")

    out_type = [jax.ShapeDtypeStruct((NSC * N_PAD, D), jnp.float32)]
    scratch = [
        pltpu.VMEM((CH,), jnp.int32),
        pltpu.VMEM((CH,), jnp.int32),
        pltpu.VMEM((CH, D), jnp.float32),
        pltpu.VMEM_SHARED((N_PAD, D), jnp.float32),
        pltpu.SemaphoreType.DMA,
    ]
    if with_deg:
        out_type.append(jax.ShapeDtypeStruct((NW, N_PAD), jnp.float32))
        scratch.append(pltpu.VMEM((N_PAD,), jnp.float32))

    cp = pltpu.CompilerParams()
    if "needs_layout_passes" in pltpu.CompilerParams.__dataclass_fields__:
        cp = dataclasses.replace(cp, needs_layout_passes=False)

    @functools.partial(pl.kernel,
                       out_type=tuple(out_type) if with_deg else out_type[0],
                       mesh=mesh, scratch_types=scratch, compiler_params=cp)
    def sc_agg(*refs):
        if with_deg:
            (table_hbm, src_hbm, dst_hbm, zrows_hbm,
             out_hbm, dout_hbm,
             src_v, dst_v, rows_v, acc, sem, hist) = refs
        else:
            (table_hbm, src_hbm, dst_hbm, zrows_hbm,
             out_hbm,
             src_v, dst_v, rows_v, acc, sem) = refs

        c = lax.axis_index("c")
        s = lax.axis_index("s")

        pltpu.sync_copy(zrows_hbm, rows_v)

        @pl.loop(0, ZITER)
        def _(j):
            pltpu.sync_copy(rows_v, acc.at[pl.ds(s * STRIPE + j * CH, CH)])

        if with_deg:
            zvec = jnp.zeros((VL,), jnp.float32)

            @pl.loop(0, N_PAD // VL)
            def _(i):
                hist[pl.ds(i * VL, VL)] = zvec

        plsc.subcore_barrier()

        ebase = (c * NSUB + s) * PER_W

        @pl.loop(0, CHUNKS)
        def _(k):
            base = ebase + k * CH
            pltpu.sync_copy(src_hbm.at[pl.ds(base, CH)], src_v)
            pltpu.sync_copy(dst_hbm.at[pl.ds(base, CH)], dst_v)
            pltpu.async_copy(table_hbm.at[src_v], rows_v, sem).wait()
            pltpu.sync_copy(rows_v, acc.at[dst_v], add=True)
            if with_deg:
                ovec = jnp.ones((VL,), jnp.float32)
                for g in range(CH // VL):
                    idx16 = dst_v[pl.ds(g * VL, VL)]
                    plsc.addupdate_scatter(hist, [idx16], ovec)

        plsc.subcore_barrier()

        @pl.loop(0, ZITER)
        def _(j):
            row = s * STRIPE + j * CH
            pltpu.sync_copy(acc.at[pl.ds(row, CH)], rows_v)
            pltpu.sync_copy(rows_v, out_hbm.at[pl.ds(c * N_PAD + row, CH)])

        if with_deg:
            pltpu.sync_copy(hist, dout_hbm.at[c * NSUB + s])

    return sc_agg


_sc_agg_deg = _make_sc_agg(with_deg=True)
_sc_agg = _make_sc_agg(with_deg=False)


def _tc_layer(psum, degT, x, w_l, w_r, b, relu: bool):
    def body(p0, p1, dT, xr, wl, wr, br, o):
        agg = p0[0] + p1[0]
        deg = jnp.sum(dT[...], axis=1, keepdims=True)
        rd = 1.0 / jnp.maximum(deg, 1.0)
        h = jnp.dot(agg * rd, wl[...], preferred_element_type=jnp.float32)
        h = h + jnp.dot(xr[...], wr[...], preferred_element_type=jnp.float32)
        h = h + br[...]
        o[...] = jnp.maximum(h, 0.0) if relu else h

    return pl.pallas_call(
        body,
        grid=(N // R,),
        in_specs=[
            pl.BlockSpec((1, R, D), lambda i: (0, i, 0)),
            pl.BlockSpec((1, R, D), lambda i: (1, i, 0)),
            pl.BlockSpec((R, NW), lambda i: (i, 0)),
            pl.BlockSpec((R, D), lambda i: (i, 0)),
            pl.BlockSpec((D, D), lambda i: (0, 0)),
            pl.BlockSpec((D, D), lambda i: (0, 0)),
            pl.BlockSpec((1, D), lambda i: (0, 0)),
        ],
        out_specs=pl.BlockSpec((R, D), lambda i: (i, 0)),
        out_shape=jax.ShapeDtypeStruct((N, D), jnp.float32),
    )(psum, psum, degT, x, w_l, w_r, b)


def kernel(x, edge_index, W1_l, W1_r, b1, W2_l, W2_r, b2):
    ei = edge_index.astype(jnp.int32)
    pad_i = jnp.arange(PAD_E, dtype=jnp.int32)
    src = jnp.concatenate([ei[0], (pad_i * 97) % N])
    dst = jnp.concatenate([ei[1], N + pad_i % (N_PAD - N)])

    zrows = jnp.zeros((CH, D), jnp.float32)

    psum1, dpart = _sc_agg_deg(x, src, dst, zrows)
    psum1 = psum1.reshape(NSC, N_PAD, D)
    degT = dpart.T
    h = _tc_layer(psum1, degT, x, W1_l, W1_r, b1.reshape(1, D), relu=True)
    psum2 = _sc_agg(h, src, dst, zrows).reshape(NSC, N_PAD, D)
    out = _tc_layer(psum2, degT, h, W2_l, W2_r, b2.reshape(1, D), relu=False)
    return out


# speedup vs baseline: 10.4165x; 1.5388x over previous
import dataclasses
import functools

import jax
import jax.numpy as jnp
from jax import lax
from jax.experimental import pallas as pl
from jax.experimental.pallas import tpu as pltpu
from jax.experimental.pallas import tpu_sc as plsc

N = 10000
E = 320000
D = 128
VL = 16

NSC = 2
NSUB = 16
NW = NSC * NSUB

GC = 64
SBC = 16
SB = GC * SBC
PER_W = 10240
NSB = PER_W // SB
WCHUNKS = PER_W // GC
E_PAD = PER_W * NW
PAD_E = E_PAD - E

N_PAD = 10240
STRIPE = N_PAD // NSUB
ZITER = STRIPE // GC

R = 1000


def _make_sc_agg(with_deg: bool):
    mesh = plsc.VectorSubcoreMesh(core_axis_name="c", subcore_axis_name="s")

    out_type = [jax.ShapeDtypeStruct((NSC * N_PAD, D), jnp.float32)]
    scratch = [
        pltpu.VMEM((SBC, GC), jnp.int32),
        pltpu.VMEM((SBC, GC), jnp.int32),
        pltpu.VMEM((GC, D), jnp.float32),
        pltpu.VMEM((GC, D), jnp.float32),
        pltpu.VMEM_SHARED((N_PAD, D), jnp.float32),
        pltpu.SemaphoreType.DMA,
        pltpu.SemaphoreType.DMA,
    ]
    if with_deg:
        out_type.append(jax.ShapeDtypeStruct((NW, N_PAD), jnp.float32))
        scratch.append(pltpu.VMEM((N_PAD,), jnp.float32))

    cp = pltpu.CompilerParams()
    if "needs_layout_passes" in pltpu.CompilerParams.__dataclass_fields__:
        cp = dataclasses.replace(cp, needs_layout_passes=False)

    @functools.partial(pl.kernel,
                       out_type=tuple(out_type) if with_deg else out_type[0],
                       mesh=mesh, scratch_types=scratch, compiler_params=cp)
    def sc_agg(*refs):
        if with_deg:
            (table_hbm, src_hbm, dst_hbm, zrows_hbm,
             out_hbm, dout_hbm,
             src_big, dst_big, rows_a, rows_b, acc, sem_a, sem_b, hist) = refs
        else:
            (table_hbm, src_hbm, dst_hbm, zrows_hbm,
             out_hbm,
             src_big, dst_big, rows_a, rows_b, acc, sem_a, sem_b) = refs

        c = lax.axis_index("c")
        s = lax.axis_index("s")
        bufs = (rows_a, rows_b)
        sems = (sem_a, sem_b)

        pltpu.sync_copy(zrows_hbm, rows_a)

        @pl.loop(0, ZITER)
        def _(j):
            pltpu.sync_copy(rows_a, acc.at[pl.ds(s * STRIPE + j * GC, GC)])

        if with_deg:
            zvec = jnp.zeros((VL,), jnp.float32)

            @pl.loop(0, N_PAD // VL)
            def _(i):
                hist[pl.ds(i * VL, VL)] = zvec

        plsc.subcore_barrier()

        cbase = (c * NSUB + s) * WCHUNKS
        ovec = jnp.ones((VL,), jnp.float32)

        def deg_count(j):
            for g in range(GC // VL):
                idx16 = dst_big[j, pl.ds(g * VL, VL)]
                plsc.addupdate_scatter(hist, [idx16], ovec)

        @pl.loop(0, NSB)
        def _(b):
            rowb = cbase + b * SBC
            pltpu.sync_copy(src_hbm.at[pl.ds(rowb, SBC)], src_big)
            pltpu.sync_copy(dst_hbm.at[pl.ds(rowb, SBC)], dst_big)

            handles = [None] * SBC
            handles[0] = pltpu.async_copy(
                table_hbm.at[src_big.at[0]], bufs[0], sems[0])
            for j in range(1, SBC):
                handles[j] = pltpu.async_copy(
                    table_hbm.at[src_big.at[j]], bufs[j % 2], sems[j % 2])
                handles[j - 1].wait()
                pltpu.sync_copy(bufs[(j - 1) % 2],
                                acc.at[dst_big.at[j - 1]], add=True)
                if with_deg:
                    deg_count(j - 1)
            handles[SBC - 1].wait()
            pltpu.sync_copy(bufs[(SBC - 1) % 2],
                            acc.at[dst_big.at[SBC - 1]], add=True)
            if with_deg:
                deg_count(SBC - 1)

        plsc.subcore_barrier()

        @pl.loop(0, ZITER)
        def _(j):
            row = s * STRIPE + j * GC
            pltpu.sync_copy(acc.at[pl.ds(row, GC)], rows_a)
            pltpu.sync_copy(rows_a, out_hbm.at[pl.ds(c * N_PAD + row, GC)])

        if with_deg:
            pltpu.sync_copy(hist, dout_hbm.at[c * NSUB + s])

    return sc_agg


_sc_agg_deg = _make_sc_agg(with_deg=True)
_sc_agg = _make_sc_agg(with_deg=False)


def _tc_layer(psum, degT, x, w_l, w_r, b, relu: bool):
    def body(p0, p1, dT, xr, wl, wr, br, o):
        agg = p0[0] + p1[0]
        deg = jnp.sum(dT[...], axis=1, keepdims=True)
        rd = 1.0 / jnp.maximum(deg, 1.0)
        h = jnp.dot(agg * rd, wl[...], preferred_element_type=jnp.float32)
        h = h + jnp.dot(xr[...], wr[...], preferred_element_type=jnp.float32)
        h = h + br[...]
        o[...] = jnp.maximum(h, 0.0) if relu else h

    return pl.pallas_call(
        body,
        grid=(N // R,),
        in_specs=[
            pl.BlockSpec((1, R, D), lambda i: (0, i, 0)),
            pl.BlockSpec((1, R, D), lambda i: (1, i, 0)),
            pl.BlockSpec((R, NW), lambda i: (i, 0)),
            pl.BlockSpec((R, D), lambda i: (i, 0)),
            pl.BlockSpec((D, D), lambda i: (0, 0)),
            pl.BlockSpec((D, D), lambda i: (0, 0)),
            pl.BlockSpec((1, D), lambda i: (0, 0)),
        ],
        out_specs=pl.BlockSpec((R, D), lambda i: (i, 0)),
        out_shape=jax.ShapeDtypeStruct((N, D), jnp.float32),
    )(psum, psum, degT, x, w_l, w_r, b)


def kernel(x, edge_index, W1_l, W1_r, b1, W2_l, W2_r, b2):
    ei = edge_index.astype(jnp.int32)
    pad_i = jnp.arange(PAD_E, dtype=jnp.int32)
    src = jnp.concatenate([ei[0], (pad_i * 97) % N]).reshape(E_PAD // GC, GC)
    dst = jnp.concatenate([ei[1], N + pad_i % (N_PAD - N)]).reshape(E_PAD // GC, GC)

    zrows = jnp.zeros((GC, D), jnp.float32)

    psum1, dpart = _sc_agg_deg(x, src, dst, zrows)
    psum1 = psum1.reshape(NSC, N_PAD, D)
    degT = dpart.T
    h = _tc_layer(psum1, degT, x, W1_l, W1_r, b1.reshape(1, D), relu=True)
    psum2 = _sc_agg(h, src, dst, zrows).reshape(NSC, N_PAD, D)
    out = _tc_layer(psum2, degT, h, W2_l, W2_r, b2.reshape(1, D), relu=False)
    return out
